# baseline (speedup 1.0000x reference)
import functools

import jax
import jax.numpy as jnp
from jax import lax
from jax.experimental import pallas as pl
from jax.experimental.pallas import tpu as pltpu
from jax.experimental.pallas import tpu_sc as plsc

_P = 0.5
_SCALE = 1.0 / (1.0 - _P)
_NC = 2
_NS = 16
_NW = _NC * _NS
_L = 16
_BLK = 8192
_DEPTH = 3
_UNROLL = 16


@functools.cache
def _build(n):
  span = (n // _NW) // 8 * 8
  covered = span * _NW
  tail = n - covered
  nb = span // _BLK
  rem = span - nb * _BLK
  chunks = [(b * _BLK, _BLK) for b in range(nb)]
  if rem:
    chunks.append((nb * _BLK, rem))

  mesh = plsc.VectorSubcoreMesh(core_axis_name="c", subcore_axis_name="s")

  @functools.partial(
      pl.kernel,
      mesh=mesh,
      out_type=jax.ShapeDtypeStruct((n,), jnp.float32),
      scratch_types=(
          [pltpu.VMEM((_BLK,), jnp.float32)] * (3 * _DEPTH)
          + [pltpu.SemaphoreType.DMA] * (3 * _DEPTH)
      ),
  )
  def dropout_k(v_hbm, m_hbm, o_hbm, *scratch):
    bufs = scratch[:3 * _DEPTH]
    sems = scratch[3 * _DEPTH:]
    vbufs = bufs[0:_DEPTH]
    mbufs = bufs[_DEPTH:2 * _DEPTH]
    obufs = bufs[2 * _DEPTH:3 * _DEPTH]
    v_sems = sems[0:_DEPTH]
    m_sems = sems[_DEPTH:2 * _DEPTH]
    o_sems = sems[2 * _DEPTH:3 * _DEPTH]
    wid = lax.axis_index("s") * _NC + lax.axis_index("c")
    base = wid * span

    def start_in(idx):
      p = idx % _DEPTH
      off = pl.multiple_of(base + chunks[idx][0], 8)
      cnt = chunks[idx][1]
      dv = pltpu.async_copy(v_hbm.at[pl.ds(off, cnt)],
                            vbufs[p].at[pl.ds(0, cnt)], v_sems[p])
      dm = pltpu.async_copy(m_hbm.at[pl.ds(off, cnt)],
                            mbufs[p].at[pl.ds(0, cnt)], m_sems[p])
      return dv, dm

    def compute(idx):
      p = idx % _DEPTH
      vb, mb, ob = vbufs[p], mbufs[p], obufs[p]
      nvec = -(-chunks[idx][1] // _L)

      @plsc.parallel_loop(0, nvec, unroll=_UNROLL)
      def _body(i):
        sl = pl.ds(i * _L, _L)
        ob[sl] = jnp.where(mb[sl] >= _P, vb[sl] * _SCALE, 0.0)

    def start_out(idx):
      p = idx % _DEPTH
      off = pl.multiple_of(base + chunks[idx][0], 8)
      cnt = chunks[idx][1]
      return pltpu.async_copy(obufs[p].at[pl.ds(0, cnt)],
                              o_hbm.at[pl.ds(off, cnt)], o_sems[p])

    in_d = {}
    out_d = {}
    for idx in range(min(_DEPTH - 1, len(chunks))):
      in_d[idx] = start_in(idx)
    for idx in range(len(chunks)):
      if idx + _DEPTH - 1 < len(chunks):
        in_d[idx + _DEPTH - 1] = start_in(idx + _DEPTH - 1)
      dv, dm = in_d.pop(idx)
      dv.wait()
      dm.wait()
      if idx >= _DEPTH:
        out_d.pop(idx - _DEPTH).wait()
      compute(idx)
      out_d[idx] = start_out(idx)
    for idx in sorted(out_d):
      out_d.pop(idx).wait()

    if tail:
      @pl.when(wid == _NW - 1)
      def _tail():
        nvec = -(-tail // _L)
        pltpu.sync_copy(v_hbm.at[pl.ds(covered, tail)],
                        vbufs[0].at[pl.ds(0, tail)])
        pltpu.sync_copy(m_hbm.at[pl.ds(covered, tail)],
                        mbufs[0].at[pl.ds(0, tail)])
        for i in range(nvec):
          sl = pl.ds(i * _L, _L)
          obufs[0][sl] = jnp.where(
              mbufs[0][sl] >= _P, vbufs[0][sl] * _SCALE, 0.0)
        pltpu.sync_copy(obufs[0].at[pl.ds(0, tail)],
                        o_hbm.at[pl.ds(covered, tail)])

  return dropout_k


def kernel(indices, values, mask_rand):
  del indices
  return _build(values.shape[0])(values, mask_rand)
